# Initial kernel scaffold; baseline (speedup 1.0000x reference)
import jax
import jax.numpy as jnp
from jax.experimental import pallas as pl


def kernel(vecs, codebook):
    raise NotImplementedError("write your pallas kernel here")


# speedup vs baseline: 1.1079x; 1.1079x over previous
import jax
import jax.numpy as jnp
from jax.experimental import pallas as pl

_B, _R, _C, _K, _S = 4, 8, 512, 256, 1024
_N = _B * _R * _C
_BN = 1024


def _vq_block(v_ref, cb_ref, z_ref, cz_ref, err_ref):
    v = v_ref[...]
    cb = cb_ref[...]
    dots = jax.lax.dot_general(
        v, cb, (((1,), (1,)), ((), ())),
        preferred_element_type=jnp.float32)
    v2 = jnp.sum(v * v, axis=1, keepdims=True)
    c2 = jnp.sum(cb * cb, axis=1)
    diffs2 = (v2 + (-2.0) * dots) + c2[None, :]
    m = jnp.min(diffs2, axis=1, keepdims=True)
    sidx = jax.lax.broadcasted_iota(jnp.int32, diffs2.shape, 1)
    z = jnp.min(jnp.where(diffs2 == m, sidx, _S), axis=1, keepdims=True)
    z_ref[...] = z
    onehot = (sidx == z).astype(jnp.float32)
    cz_ref[...] = jax.lax.dot_general(
        onehot, cb, (((1,), (0,)), ((), ())),
        preferred_element_type=jnp.float32,
        precision=jax.lax.Precision.HIGHEST)
    partial = jnp.sum(jnp.maximum(m, 0.0))

    @pl.when(pl.program_id(0) == 0)
    def _init():
        err_ref[...] = jnp.zeros_like(err_ref)

    err_ref[...] += partial


def _vq_tc(vf, codebook):
    return pl.pallas_call(
        _vq_block,
        grid=(_N // _BN,),
        in_specs=[
            pl.BlockSpec((_BN, _K), lambda i: (i, 0)),
            pl.BlockSpec((_S, _K), lambda i: (0, 0)),
        ],
        out_specs=[
            pl.BlockSpec((_BN, 1), lambda i: (i, 0)),
            pl.BlockSpec((_BN, _K), lambda i: (i, 0)),
            pl.BlockSpec((8, 128), lambda i: (0, 0)),
        ],
        out_shape=[
            jax.ShapeDtypeStruct((_N, 1), jnp.int32),
            jax.ShapeDtypeStruct((_N, _K), jnp.float32),
            jax.ShapeDtypeStruct((8, 128), jnp.float32),
        ],
    )(vf, codebook)


def kernel(vecs, codebook):
    orig_dtype = vecs.dtype
    vf = vecs.astype(jnp.float32).reshape(_N, _K)
    z_col, cz, err_acc = _vq_tc(vf, codebook)
    z = z_col.reshape(_B, _R, _C)
    vecs_hat = cz.reshape(_B, _R, _C, _K).astype(orig_dtype)
    l = (err_acc[0, 0] / _N).astype(jnp.float32)
    return (vecs_hat, z, l, l)


# speedup vs baseline: 1.9458x; 1.7563x over previous
import jax
import jax.numpy as jnp
from jax.experimental import pallas as pl

_B, _R, _C, _K, _S = 4, 8, 512, 256, 1024
_N = _B * _R * _C
_BN = 1024


def _vq_block(v_ref, cb_ref, z_ref, cz_ref, err_ref):
    v = v_ref[...]
    cb = cb_ref[...]
    dots = jax.lax.dot_general(
        v, cb, (((1,), (1,)), ((), ())),
        preferred_element_type=jnp.float32)
    v2 = jnp.sum(v * v, axis=1, keepdims=True)
    c2 = jnp.sum(cb * cb, axis=1)
    diffs2 = (v2 + (-2.0) * dots) + c2[None, :]
    m = jnp.min(diffs2, axis=1, keepdims=True)
    sidx = jax.lax.broadcasted_iota(jnp.int32, diffs2.shape, 1)
    z = jnp.min(jnp.where(diffs2 == m, sidx, _S), axis=1, keepdims=True)
    z_ref[...] = z
    onehot = (sidx == z).astype(jnp.float32)
    cz_ref[...] = jax.lax.dot_general(
        onehot, cb, (((1,), (0,)), ((), ())),
        preferred_element_type=jnp.float32)
    partial = jnp.sum(jnp.maximum(m, 0.0))

    @pl.when(pl.program_id(0) == 0)
    def _init():
        err_ref[...] = jnp.zeros_like(err_ref)

    err_ref[...] += partial


def _vq_tc(vf, codebook):
    return pl.pallas_call(
        _vq_block,
        grid=(_N // _BN,),
        in_specs=[
            pl.BlockSpec((_BN, _K), lambda i: (i, 0)),
            pl.BlockSpec((_S, _K), lambda i: (0, 0)),
        ],
        out_specs=[
            pl.BlockSpec((_BN, 1), lambda i: (i, 0)),
            pl.BlockSpec((_BN, _K), lambda i: (i, 0)),
            pl.BlockSpec((8, 128), lambda i: (0, 0)),
        ],
        out_shape=[
            jax.ShapeDtypeStruct((_N, 1), jnp.int32),
            jax.ShapeDtypeStruct((_N, _K), jnp.float32),
            jax.ShapeDtypeStruct((8, 128), jnp.float32),
        ],
    )(vf, codebook)


def kernel(vecs, codebook):
    orig_dtype = vecs.dtype
    vf = vecs.astype(jnp.float32).reshape(_N, _K)
    z_col, cz, err_acc = _vq_tc(vf, codebook)
    z = z_col.reshape(_B, _R, _C)
    vecs_hat = cz.reshape(_B, _R, _C, _K).astype(orig_dtype)
    l = (err_acc[0, 0] / _N).astype(jnp.float32)
    return (vecs_hat, z, l, l)


# speedup vs baseline: 2.1031x; 1.0808x over previous
import jax
import jax.numpy as jnp
from jax.experimental import pallas as pl

_B, _R, _C, _K, _S = 4, 8, 512, 256, 1024
_N = _B * _R * _C
_BN = 1024


def _vq_block(v_ref, cb_ref, z_ref, cz_ref, err_ref):
    v = v_ref[...]
    cb = cb_ref[...]
    dots2 = jax.lax.dot_general(
        v * (-2.0), cb, (((1,), (1,)), ((), ())),
        preferred_element_type=jnp.float32)
    v2 = jnp.sum(v * v, axis=1, keepdims=True)
    c2 = jnp.sum(cb * cb, axis=1)
    diffs2 = (v2 + dots2) + c2[None, :]
    m = jnp.min(diffs2, axis=1, keepdims=True)
    sidxf = jax.lax.broadcasted_iota(
        jnp.int32, diffs2.shape, 1).astype(jnp.float32)
    zf = jnp.min(jnp.where(diffs2 == m, sidxf, float(_S)),
                 axis=1, keepdims=True)
    z_ref[...] = zf.astype(jnp.int32)
    onehot = (sidxf == zf).astype(jnp.float32)
    cz_ref[...] = jax.lax.dot_general(
        onehot, cb, (((1,), (0,)), ((), ())),
        preferred_element_type=jnp.float32)
    partial = jnp.sum(jnp.maximum(m, 0.0))

    @pl.when(pl.program_id(0) == 0)
    def _init():
        err_ref[...] = jnp.zeros_like(err_ref)

    err_ref[...] += partial


def _vq_tc(vf, codebook):
    return pl.pallas_call(
        _vq_block,
        grid=(_N // _BN,),
        in_specs=[
            pl.BlockSpec((_BN, _K), lambda i: (i, 0)),
            pl.BlockSpec((_S, _K), lambda i: (0, 0)),
        ],
        out_specs=[
            pl.BlockSpec((_BN, 1), lambda i: (i, 0)),
            pl.BlockSpec((_BN, _K), lambda i: (i, 0)),
            pl.BlockSpec((8, 128), lambda i: (0, 0)),
        ],
        out_shape=[
            jax.ShapeDtypeStruct((_N, 1), jnp.int32),
            jax.ShapeDtypeStruct((_N, _K), jnp.float32),
            jax.ShapeDtypeStruct((8, 128), jnp.float32),
        ],
    )(vf, codebook)


def kernel(vecs, codebook):
    orig_dtype = vecs.dtype
    vf = vecs.astype(jnp.float32).reshape(_N, _K)
    z_col, cz, err_acc = _vq_tc(vf, codebook)
    z = z_col.reshape(_B, _R, _C)
    vecs_hat = cz.reshape(_B, _R, _C, _K).astype(orig_dtype)
    l = (err_acc[0, 0] / _N).astype(jnp.float32)
    return (vecs_hat, z, l, l)


# speedup vs baseline: 2.2152x; 1.0533x over previous
import jax
import jax.numpy as jnp
from jax.experimental import pallas as pl

_B, _R, _C, _K, _S = 4, 8, 512, 256, 1024
_N = _B * _R * _C
_BN = 2048


def _vq_block(v_ref, cb_ref, z_ref, cz_ref, err_ref):
    v = v_ref[...]
    cb = cb_ref[...]
    dots2 = jax.lax.dot_general(
        v * (-2.0), cb, (((1,), (1,)), ((), ())),
        preferred_element_type=jnp.float32)
    v2 = jnp.sum(v * v, axis=1, keepdims=True)
    c2 = jnp.sum(cb * cb, axis=1)
    diffs2 = (v2 + dots2) + c2[None, :]
    m = jnp.min(diffs2, axis=1, keepdims=True)
    sidxf = jax.lax.broadcasted_iota(
        jnp.int32, diffs2.shape, 1).astype(jnp.float32)
    zf = jnp.min(jnp.where(diffs2 == m, sidxf, float(_S)),
                 axis=1, keepdims=True)
    z_ref[...] = zf.astype(jnp.int32)
    onehot = (sidxf == zf).astype(jnp.float32)
    cz_ref[...] = jax.lax.dot_general(
        onehot, cb, (((1,), (0,)), ((), ())),
        preferred_element_type=jnp.float32)
    partial = jnp.sum(jnp.maximum(m, 0.0))

    @pl.when(pl.program_id(0) == 0)
    def _init():
        err_ref[...] = jnp.zeros_like(err_ref)

    err_ref[...] += partial


def _vq_tc(vf, codebook):
    return pl.pallas_call(
        _vq_block,
        grid=(_N // _BN,),
        in_specs=[
            pl.BlockSpec((_BN, _K), lambda i: (i, 0)),
            pl.BlockSpec((_S, _K), lambda i: (0, 0)),
        ],
        out_specs=[
            pl.BlockSpec((_BN, 1), lambda i: (i, 0)),
            pl.BlockSpec((_BN, _K), lambda i: (i, 0)),
            pl.BlockSpec((8, 128), lambda i: (0, 0)),
        ],
        out_shape=[
            jax.ShapeDtypeStruct((_N, 1), jnp.int32),
            jax.ShapeDtypeStruct((_N, _K), jnp.float32),
            jax.ShapeDtypeStruct((8, 128), jnp.float32),
        ],
    )(vf, codebook)


def kernel(vecs, codebook):
    orig_dtype = vecs.dtype
    vf = vecs.astype(jnp.float32).reshape(_N, _K)
    z_col, cz, err_acc = _vq_tc(vf, codebook)
    z = z_col.reshape(_B, _R, _C)
    vecs_hat = cz.reshape(_B, _R, _C, _K).astype(orig_dtype)
    l = (err_acc[0, 0] / _N).astype(jnp.float32)
    return (vecs_hat, z, l, l)


# speedup vs baseline: 2.2440x; 1.0130x over previous
import jax
import jax.numpy as jnp
from jax.experimental import pallas as pl

_B, _R, _C, _K, _S = 4, 8, 512, 256, 1024
_N = _B * _R * _C
_BN = 4096


def _vq_block(v_ref, cb_ref, z_ref, cz_ref, err_ref):
    v = v_ref[...]
    cb = cb_ref[...]
    dots2 = jax.lax.dot_general(
        v * (-2.0), cb, (((1,), (1,)), ((), ())),
        preferred_element_type=jnp.float32)
    v2 = jnp.sum(v * v, axis=1, keepdims=True)
    c2 = jnp.sum(cb * cb, axis=1)
    diffs2 = (v2 + dots2) + c2[None, :]
    m = jnp.min(diffs2, axis=1, keepdims=True)
    sidxf = jax.lax.broadcasted_iota(
        jnp.int32, diffs2.shape, 1).astype(jnp.float32)
    zf = jnp.min(jnp.where(diffs2 == m, sidxf, float(_S)),
                 axis=1, keepdims=True)
    z_ref[...] = zf.astype(jnp.int32)
    onehot = (sidxf == zf).astype(jnp.float32)
    cz_ref[...] = jax.lax.dot_general(
        onehot, cb, (((1,), (0,)), ((), ())),
        preferred_element_type=jnp.float32)
    partial = jnp.sum(jnp.maximum(m, 0.0))

    @pl.when(pl.program_id(0) == 0)
    def _init():
        err_ref[...] = jnp.zeros_like(err_ref)

    err_ref[...] += partial


def _vq_tc(vf, codebook):
    return pl.pallas_call(
        _vq_block,
        grid=(_N // _BN,),
        in_specs=[
            pl.BlockSpec((_BN, _K), lambda i: (i, 0)),
            pl.BlockSpec((_S, _K), lambda i: (0, 0)),
        ],
        out_specs=[
            pl.BlockSpec((_BN, 1), lambda i: (i, 0)),
            pl.BlockSpec((_BN, _K), lambda i: (i, 0)),
            pl.BlockSpec((8, 128), lambda i: (0, 0)),
        ],
        out_shape=[
            jax.ShapeDtypeStruct((_N, 1), jnp.int32),
            jax.ShapeDtypeStruct((_N, _K), jnp.float32),
            jax.ShapeDtypeStruct((8, 128), jnp.float32),
        ],
    )(vf, codebook)


def kernel(vecs, codebook):
    orig_dtype = vecs.dtype
    vf = vecs.astype(jnp.float32).reshape(_N, _K)
    z_col, cz, err_acc = _vq_tc(vf, codebook)
    z = z_col.reshape(_B, _R, _C)
    vecs_hat = cz.reshape(_B, _R, _C, _K).astype(orig_dtype)
    l = (err_acc[0, 0] / _N).astype(jnp.float32)
    return (vecs_hat, z, l, l)
